# baseline (speedup 1.0000x reference)
import jax
import jax.numpy as jnp
from jax import lax
from jax.experimental import pallas as pl
from jax.experimental.pallas import tpu as pltpu
from jax.experimental.pallas import tpu_sc as plsc

N_SPIKES = 65536
N_UNITS = 256
D = 128
C_TOT = 10
N_CAND = 3

NC = 2
NS = 16
NW = NC * NS

CHUNKS = 4
N_CHUNK = N_SPIKES // CHUNKS
SPT = N_CHUNK // NW
S_SUB = 256
N_SUB = SPT // S_SUB

BLK_A = 4096
BLK_C = 4096

_DEF = lax.Precision.DEFAULT


def _dot3(a, b, dims):
    a_hi = a.astype(jnp.bfloat16).astype(jnp.float32)
    a_lo = a - a_hi
    b_hi = b.astype(jnp.bfloat16).astype(jnp.float32)
    b_lo = b - b_hi
    dg = lambda x, y: lax.dot_general(x, y, dims,
                                      preferred_element_type=jnp.float32,
                                      precision=_DEF)
    return dg(a_hi, b_hi) + (dg(a_hi, b_lo) + dg(a_lo, b_hi))


def _stage_a_body(f_ref, m_ref, lp_ref, logits_ref, rowsq_ref):
    f = f_ref[...]
    m = m_ref[...]
    cdims = (((1,), (1,)), ((), ()))
    prod = _dot3(f, m, cdims)
    ones = jnp.ones((8, D), jnp.float32)
    dg = lambda x, y: lax.dot_general(x, y, cdims,
                                      preferred_element_type=jnp.float32,
                                      precision=_DEF)
    msq = m * m
    msq_hi = msq.astype(jnp.bfloat16).astype(jnp.float32)
    musq = (dg(ones, msq_hi) + dg(ones, msq - msq_hi))[0:1]
    logits_ref[...] = prod + lp_ref[...] - 0.5 * musq
    fsq = f * f
    fsq_hi = fsq.astype(jnp.bfloat16).astype(jnp.float32)
    rsq = (dg(ones, fsq_hi) + dg(ones, fsq - fsq_hi))[0:1]
    rowsq_ref[...] = -0.5 * rsq


def _stage_a(features, means, lp2d, chunk):
    grid = (N_CHUNK // BLK_A,)
    off = chunk * (N_CHUNK // BLK_A)
    return pl.pallas_call(
        _stage_a_body,
        grid=grid,
        in_specs=[
            pl.BlockSpec((BLK_A, D), lambda i: (i + off, 0)),
            pl.BlockSpec((N_UNITS, D), lambda i: (0, 0)),
            pl.BlockSpec((1, N_UNITS), lambda i: (0, 0)),
        ],
        out_specs=[
            pl.BlockSpec((BLK_A, N_UNITS), lambda i: (i, 0)),
            pl.BlockSpec((1, BLK_A), lambda i: (0, i)),
        ],
        out_shape=[
            jax.ShapeDtypeStruct((N_CHUNK, N_UNITS), jnp.float32),
            jax.ShapeDtypeStruct((1, N_CHUNK), jnp.float32),
        ],
    )(features, means, lp2d)


def _make_stage_b_body(chunk):
    goff = chunk * N_CHUNK

    def _stage_b_body(logits_h, cand_h, rowsq_h,
                      tv_h, tc_h, q_h, ssum_h, ma_h, np_h,
                      logits_v, cand_v, rowsq_v,
                      tv_v, tc_v, q_v, ssum_v, ma_v, nacc_v, npt_v):
        wid = lax.axis_index("s") * NC + lax.axis_index("c")
        tile_base = wid * SPT
        lane = lax.iota(jnp.int32, 16)
        zero16 = jnp.zeros((16,), jnp.float32)
        lane_u = lane * N_UNITS

        for j in range(16 * N_UNITS // 16):
            nacc_v[pl.ds(j * 16, 16)] = zero16

        def sub(s, carry):
            base = tile_base + s * S_SUB
            cols = pl.ds(base, S_SUB)
            pltpu.sync_copy(logits_h.at[pl.ds(base, S_SUB)], logits_v)
            pltpu.sync_copy(cand_h.at[:, pl.ds(goff + base, S_SUB)], cand_v)
            pltpu.sync_copy(rowsq_h.at[:, cols], rowsq_v)
            for g in range(S_SUB // 16):
                row = lane + g * 16
                rsq = rowsq_v[0, pl.ds(g * 16, 16)]
                neg = jnp.full((16,), -3e38, jnp.float32)
                zi = jnp.zeros((16,), jnp.int32)
                m1, m2, m3 = neg, neg, neg
                i1, i2, i3 = zi, zi, zi
                vals = []
                cids = []
                for c in range(C_TOT):
                    cc = plsc.load_gather(
                        cand_v, [jnp.full((16,), c, jnp.int32), row])
                    v = plsc.load_gather(logits_v, [row, cc])
                    vals.append(v)
                    cids.append(cc)
                    gt1 = v > m1
                    gt2 = v > m2
                    gt3 = v > m3
                    m3 = jnp.where(gt2, m2, jnp.where(gt3, v, m3))
                    i3 = jnp.where(gt2, i2, jnp.where(gt3, cc, i3))
                    m2 = jnp.where(gt1, m1, jnp.where(gt2, v, m2))
                    i2 = jnp.where(gt1, i1, jnp.where(gt2, cc, i2))
                    m1 = jnp.where(gt1, v, m1)
                    i1 = jnp.where(gt1, cc, i1)
                ssum = zero16
                es = []
                for c in range(C_TOT):
                    e = jnp.exp(vals[c] - m1)
                    es.append(e)
                    ssum = ssum + e
                rs = 1.0 / ssum
                g16 = pl.ds(g * 16, 16)
                for c in range(C_TOT):
                    qc = es[c] * rs
                    q_v[c, g16] = qc
                    plsc.addupdate_scatter(nacc_v, [lane_u + cids[c]], qc)
                ssum_v[0, g16] = ssum
                ma_v[0, g16] = m1 + rsq
                for j, (mj, ij) in enumerate(((m1, i1), (m2, i2), (m3, i3))):
                    tv_v[j, g16] = mj + rsq
                    tc_v[j, g16] = ij
            pltpu.sync_copy(tv_v, tv_h.at[:, cols])
            pltpu.sync_copy(tc_v, tc_h.at[:, cols])
            pltpu.sync_copy(q_v, q_h.at[:, cols])
            pltpu.sync_copy(ssum_v, ssum_h.at[:, cols])
            pltpu.sync_copy(ma_v, ma_h.at[:, cols])
            return carry

        lax.fori_loop(0, N_SUB, sub, 0)

        for j in range(N_UNITS // 16):
            acc = zero16
            for r in range(16):
                acc = acc + nacc_v[pl.ds(r * N_UNITS + j * 16, 16)]
            npt_v[0, pl.ds(j * 16, 16)] = acc
        pltpu.sync_copy(npt_v, np_h.at[pl.ds(wid, 1)])

    return _stage_b_body


def _stage_b(logits, candT, rowsq2, chunk):
    mesh = plsc.VectorSubcoreMesh(core_axis_name="c", subcore_axis_name="s",
                                  num_cores=NC, num_subcores=NS)
    f = pl.kernel(
        _make_stage_b_body(chunk),
        out_type=[
            jax.ShapeDtypeStruct((N_CAND, N_CHUNK), jnp.float32),
            jax.ShapeDtypeStruct((N_CAND, N_CHUNK), jnp.int32),
            jax.ShapeDtypeStruct((C_TOT, N_CHUNK), jnp.float32),
            jax.ShapeDtypeStruct((1, N_CHUNK), jnp.float32),
            jax.ShapeDtypeStruct((1, N_CHUNK), jnp.float32),
            jax.ShapeDtypeStruct((NW, N_UNITS), jnp.float32),
        ],
        mesh=mesh,
        compiler_params=pltpu.CompilerParams(needs_layout_passes=False),
        scratch_types=[
            pltpu.VMEM((S_SUB, N_UNITS), jnp.float32),
            pltpu.VMEM((C_TOT, S_SUB), jnp.int32),
            pltpu.VMEM((1, S_SUB), jnp.float32),
            pltpu.VMEM((N_CAND, S_SUB), jnp.float32),
            pltpu.VMEM((N_CAND, S_SUB), jnp.int32),
            pltpu.VMEM((C_TOT, S_SUB), jnp.float32),
            pltpu.VMEM((1, S_SUB), jnp.float32),
            pltpu.VMEM((1, S_SUB), jnp.float32),
            pltpu.VMEM((16 * N_UNITS,), jnp.float32),
            pltpu.VMEM((1, N_UNITS), jnp.float32),
        ],
    )
    return f(logits, candT, rowsq2)


def _stage_c_body(q_ref, cand_ref, f_ref, ssum_ref, ma_ref, np_ref,
                  m_ref, n_ref, e_ref, m_acc, e_acc):
    i = pl.program_id(0)
    nsteps = pl.num_programs(0)

    @pl.when(i == 0)
    def _init():
        m_acc[...] = jnp.zeros_like(m_acc)
        e_acc[0] = 0.0

    SB = 128
    units = lax.broadcasted_iota(jnp.int32, (N_UNITS, SB), 0).astype(jnp.bfloat16)
    cand16 = cand_ref[...].astype(jnp.bfloat16)
    q16 = q_ref[...].astype(jnp.bfloat16)
    for sb in range(BLK_C // SB):
        cols = pl.ds(sb * SB, SB)
        wt = jnp.zeros((N_UNITS, SB), jnp.bfloat16)
        for c in range(C_TOT):
            cb = jnp.broadcast_to(lax.slice(cand16, (c, sb * SB),
                                            (c + 1, sb * SB + SB)),
                                  (N_UNITS, SB))
            qb = jnp.broadcast_to(lax.slice(q16, (c, sb * SB),
                                            (c + 1, sb * SB + SB)),
                                  (N_UNITS, SB))
            wt = wt + jnp.where(units == cb, qb, jnp.bfloat16(0))
        m_acc[...] += lax.dot_general(wt, f_ref[cols, :].astype(jnp.bfloat16),
                                      (((1,), (0,)), ((), ())),
                                      preferred_element_type=jnp.float32,
                                      precision=_DEF)
    e_acc[0] += jnp.sum(jnp.log(ssum_ref[...]) + ma_ref[...])

    @pl.when(i == nsteps - 1)
    def _fin():
        m_ref[...] = m_acc[...]
        n_ref[...] = jnp.sum(np_ref[...], axis=0, keepdims=True)
        e_ref[...] = jnp.broadcast_to(e_acc[0] * (1.0 / N_SPIKES), (1, 1))


def _stage_c(qT, candT, features, ssum2, ma2, npart, chunk):
    grid = (N_CHUNK // BLK_C,)
    offc = chunk * (N_CHUNK // BLK_C)
    return pl.pallas_call(
        _stage_c_body,
        grid=grid,
        in_specs=[
            pl.BlockSpec((C_TOT, BLK_C), lambda i: (0, i)),
            pl.BlockSpec((C_TOT, BLK_C), lambda i: (0, i + offc)),
            pl.BlockSpec((BLK_C, D), lambda i: (i + offc, 0)),
            pl.BlockSpec((1, BLK_C), lambda i: (0, i)),
            pl.BlockSpec((1, BLK_C), lambda i: (0, i)),
            pl.BlockSpec((NW, N_UNITS), lambda i: (0, 0)),
        ],
        out_specs=[
            pl.BlockSpec((N_UNITS, D), lambda i: (0, 0)),
            pl.BlockSpec((1, N_UNITS), lambda i: (0, 0)),
            pl.BlockSpec((1, 1), lambda i: (0, 0)),
        ],
        out_shape=[
            jax.ShapeDtypeStruct((N_UNITS, D), jnp.float32),
            jax.ShapeDtypeStruct((1, N_UNITS), jnp.float32),
            jax.ShapeDtypeStruct((1, 1), jnp.float32),
        ],
        scratch_shapes=[
            pltpu.VMEM((N_UNITS, D), jnp.float32),
            pltpu.SMEM((1,), jnp.float32),
        ],
    )(qT, candT, features, ssum2, ma2, npart)


def kernel(features, candidates, means, log_proportions):
    candT = candidates.astype(jnp.int32).T
    lp2d = log_proportions.reshape(1, N_UNITS)
    tvs, tcs, ms, ns, es = [], [], [], [], []
    for h in range(CHUNKS):
        logits, rowsq2 = _stage_a(features, means, lp2d, h)
        tvT, tcT, qT, ssum2, ma2, npart = _stage_b(logits, candT, rowsq2, h)
        m_h, n_h, e_h = _stage_c(qT, candT, features, ssum2, ma2, npart, h)
        tvs.append(tvT)
        tcs.append(tcT)
        ms.append(m_h)
        ns.append(n_h)
        es.append(e_h)
    tvT = jnp.concatenate(tvs, axis=1)
    tcT = jnp.concatenate(tcs, axis=1)
    m_stat = sum(ms[1:], ms[0])
    n2d = sum(ns[1:], ns[0])
    e11 = sum(es[1:], es[0])
    return (tvT.T, tcT.T, n2d.reshape(N_UNITS), m_stat, e11.reshape(()))


# speedup vs baseline: 1.1812x; 1.1812x over previous
import jax
import jax.numpy as jnp
from jax import lax
from jax.experimental import pallas as pl
from jax.experimental.pallas import tpu as pltpu
from jax.experimental.pallas import tpu_sc as plsc

N_SPIKES = 65536
N_UNITS = 256
D = 128
C_TOT = 10
N_CAND = 3

NC = 2
NS = 16
NW = NC * NS

CHUNKS = 4
N_CHUNK = N_SPIKES // CHUNKS
SPT = N_CHUNK // NW
S_SUB = 128
N_SUB = SPT // S_SUB

BLK_A = 4096
BLK_C = 4096

_DEF = lax.Precision.DEFAULT


def _dot3(a, b, dims):
    a_hi = a.astype(jnp.bfloat16).astype(jnp.float32)
    a_lo = a - a_hi
    b_hi = b.astype(jnp.bfloat16).astype(jnp.float32)
    b_lo = b - b_hi
    dg = lambda x, y: lax.dot_general(x, y, dims,
                                      preferred_element_type=jnp.float32,
                                      precision=_DEF)
    return dg(a_hi, b_hi) + (dg(a_hi, b_lo) + dg(a_lo, b_hi))


def _stage_a_body(f_ref, m_ref, lp_ref, logits_ref, rowsq_ref):
    f = f_ref[...]
    m = m_ref[...]
    cdims = (((1,), (1,)), ((), ()))
    prod = _dot3(f, m, cdims)
    ones = jnp.ones((8, D), jnp.float32)
    dg = lambda x, y: lax.dot_general(x, y, cdims,
                                      preferred_element_type=jnp.float32,
                                      precision=_DEF)
    msq = m * m
    msq_hi = msq.astype(jnp.bfloat16).astype(jnp.float32)
    musq = (dg(ones, msq_hi) + dg(ones, msq - msq_hi))[0:1]
    logits_ref[...] = prod + lp_ref[...] - 0.5 * musq
    fsq = f * f
    fsq_hi = fsq.astype(jnp.bfloat16).astype(jnp.float32)
    rsq = (dg(ones, fsq_hi) + dg(ones, fsq - fsq_hi))[0:1]
    rowsq_ref[...] = -0.5 * rsq


def _stage_a(features, means, lp2d, chunk):
    grid = (N_CHUNK // BLK_A,)
    off = chunk * (N_CHUNK // BLK_A)
    return pl.pallas_call(
        _stage_a_body,
        grid=grid,
        in_specs=[
            pl.BlockSpec((BLK_A, D), lambda i: (i + off, 0)),
            pl.BlockSpec((N_UNITS, D), lambda i: (0, 0)),
            pl.BlockSpec((1, N_UNITS), lambda i: (0, 0)),
        ],
        out_specs=[
            pl.BlockSpec((BLK_A, N_UNITS), lambda i: (i, 0)),
            pl.BlockSpec((1, BLK_A), lambda i: (0, i)),
        ],
        out_shape=[
            jax.ShapeDtypeStruct((N_CHUNK, N_UNITS), jnp.float32),
            jax.ShapeDtypeStruct((1, N_CHUNK), jnp.float32),
        ],
    )(features, means, lp2d)


def _make_stage_b_body(chunk):
    goff = chunk * N_CHUNK

    def _stage_b_body(logits_h, cand_h, rowsq_h,
                      tv_h, tc_h, q_h, ssum_h, ma_h, np_h,
                      logits_v0, cand_v0, rowsq_v0,
                      logits_v1, cand_v1, rowsq_v1,
                      tv_v0, tc_v0, q_v0, ssum_v0, ma_v0,
                      tv_v1, tc_v1, q_v1, ssum_v1, ma_v1,
                      nacc_v, npt_v, sin0, sin1, sout0, sout1):
        wid = lax.axis_index("s") * NC + lax.axis_index("c")
        tile_base = wid * SPT
        lane = lax.iota(jnp.int32, 16)
        zero16 = jnp.zeros((16,), jnp.float32)
        lane_u = lane * N_UNITS

        logits_v = (logits_v0, logits_v1)
        cand_v = (cand_v0, cand_v1)
        rowsq_v = (rowsq_v0, rowsq_v1)
        tv_v = (tv_v0, tv_v1)
        tc_v = (tc_v0, tc_v1)
        q_v = (q_v0, q_v1)
        ssum_v = (ssum_v0, ssum_v1)
        ma_v = (ma_v0, ma_v1)
        sin = (sin0, sin1)
        sout = (sout0, sout1)

        def in_copies(s, b):
            base = tile_base + s * S_SUB
            return (
                pltpu.make_async_copy(logits_h.at[pl.ds(base, S_SUB)],
                                      logits_v[b], sin[b]),
                pltpu.make_async_copy(cand_h.at[:, pl.ds(goff + base, S_SUB)],
                                      cand_v[b], sin[b]),
                pltpu.make_async_copy(rowsq_h.at[:, pl.ds(base, S_SUB)],
                                      rowsq_v[b], sin[b]),
            )

        def out_copies(s, b):
            base = tile_base + s * S_SUB
            cols = pl.ds(base, S_SUB)
            return (
                pltpu.make_async_copy(tv_v[b], tv_h.at[:, cols], sout[b]),
                pltpu.make_async_copy(tc_v[b], tc_h.at[:, cols], sout[b]),
                pltpu.make_async_copy(q_v[b], q_h.at[:, cols], sout[b]),
                pltpu.make_async_copy(ssum_v[b], ssum_h.at[:, cols], sout[b]),
                pltpu.make_async_copy(ma_v[b], ma_h.at[:, cols], sout[b]),
            )

        for cp in in_copies(0, 0):
            cp.start()

        for j in range(16 * N_UNITS // 16):
            nacc_v[pl.ds(j * 16, 16)] = zero16

        def body2(k, carry):
            s0 = 2 * k
            for b in range(2):
                s = s0 + b
                compute_sub(s, b)
            for b in range(2):
                for cp in out_copies(s0 + b, b):
                    cp.wait()
            return carry

        def compute_sub(s, b):
            for cp in in_copies(s, b):
                cp.wait()

            @pl.when(s + 1 < N_SUB)
            def _prefetch():
                for cp in in_copies(s + 1, 1 - b):
                    cp.start()

            for g in range(S_SUB // 16):
                row = lane + g * 16
                rsq = rowsq_v[b][0, pl.ds(g * 16, 16)]
                neg = jnp.full((16,), -3e38, jnp.float32)
                zi = jnp.zeros((16,), jnp.int32)
                m1, m2, m3 = neg, neg, neg
                i1, i2, i3 = zi, zi, zi
                vals = []
                cids = []
                for c in range(C_TOT):
                    cc = plsc.load_gather(
                        cand_v[b], [jnp.full((16,), c, jnp.int32), row])
                    v = plsc.load_gather(logits_v[b], [row, cc])
                    vals.append(v)
                    cids.append(cc)
                    gt1 = v > m1
                    gt2 = v > m2
                    gt3 = v > m3
                    m3 = jnp.where(gt2, m2, jnp.where(gt3, v, m3))
                    i3 = jnp.where(gt2, i2, jnp.where(gt3, cc, i3))
                    m2 = jnp.where(gt1, m1, jnp.where(gt2, v, m2))
                    i2 = jnp.where(gt1, i1, jnp.where(gt2, cc, i2))
                    m1 = jnp.where(gt1, v, m1)
                    i1 = jnp.where(gt1, cc, i1)
                ssum = zero16
                es = []
                for c in range(C_TOT):
                    e = jnp.exp(vals[c] - m1)
                    es.append(e)
                    ssum = ssum + e
                rs = 1.0 / ssum
                g16 = pl.ds(g * 16, 16)
                for c in range(C_TOT):
                    qc = es[c] * rs
                    q_v[b][c, g16] = qc
                    plsc.addupdate_scatter(nacc_v, [lane_u + cids[c]], qc)
                ssum_v[b][0, g16] = ssum
                ma_v[b][0, g16] = m1 + rsq
                for j, (mj, ij) in enumerate(((m1, i1), (m2, i2), (m3, i3))):
                    tv_v[b][j, g16] = mj + rsq
                    tc_v[b][j, g16] = ij
            for cp in out_copies(s, b):
                cp.start()

        lax.fori_loop(0, N_SUB // 2, body2, 0)

        for j in range(N_UNITS // 16):
            acc = zero16
            for r in range(16):
                acc = acc + nacc_v[pl.ds(r * N_UNITS + j * 16, 16)]
            npt_v[0, pl.ds(j * 16, 16)] = acc
        pltpu.sync_copy(npt_v, np_h.at[pl.ds(wid, 1)])

    return _stage_b_body


def _stage_b(logits, candT, rowsq2, chunk):
    mesh = plsc.VectorSubcoreMesh(core_axis_name="c", subcore_axis_name="s",
                                  num_cores=NC, num_subcores=NS)
    f = pl.kernel(
        _make_stage_b_body(chunk),
        out_type=[
            jax.ShapeDtypeStruct((N_CAND, N_CHUNK), jnp.float32),
            jax.ShapeDtypeStruct((N_CAND, N_CHUNK), jnp.int32),
            jax.ShapeDtypeStruct((C_TOT, N_CHUNK), jnp.float32),
            jax.ShapeDtypeStruct((1, N_CHUNK), jnp.float32),
            jax.ShapeDtypeStruct((1, N_CHUNK), jnp.float32),
            jax.ShapeDtypeStruct((NW, N_UNITS), jnp.float32),
        ],
        mesh=mesh,
        compiler_params=pltpu.CompilerParams(needs_layout_passes=False),
        scratch_types=(
            [pltpu.VMEM((S_SUB, N_UNITS), jnp.float32),
             pltpu.VMEM((C_TOT, S_SUB), jnp.int32),
             pltpu.VMEM((1, S_SUB), jnp.float32)] * 2 +
            [pltpu.VMEM((N_CAND, S_SUB), jnp.float32),
             pltpu.VMEM((N_CAND, S_SUB), jnp.int32),
             pltpu.VMEM((C_TOT, S_SUB), jnp.float32),
             pltpu.VMEM((1, S_SUB), jnp.float32),
             pltpu.VMEM((1, S_SUB), jnp.float32)] * 2 +
            [pltpu.VMEM((16 * N_UNITS,), jnp.float32),
             pltpu.VMEM((1, N_UNITS), jnp.float32)] +
            [pltpu.SemaphoreType.DMA] * 4
        ),
    )
    return f(logits, candT, rowsq2)


def _stage_c_body(q_ref, cand_ref, f_ref, ssum_ref, ma_ref, np_ref,
                  m_ref, n_ref, e_ref, m_acc, e_acc):
    i = pl.program_id(0)
    nsteps = pl.num_programs(0)

    @pl.when(i == 0)
    def _init():
        m_acc[...] = jnp.zeros_like(m_acc)
        e_acc[0] = 0.0

    SB = 128
    units = lax.broadcasted_iota(jnp.int32, (N_UNITS, SB), 0).astype(jnp.bfloat16)
    cand16 = cand_ref[...].astype(jnp.bfloat16)
    q16 = q_ref[...].astype(jnp.bfloat16)
    for sb in range(BLK_C // SB):
        cols = pl.ds(sb * SB, SB)
        wt = jnp.zeros((N_UNITS, SB), jnp.bfloat16)
        for c in range(C_TOT):
            cb = jnp.broadcast_to(lax.slice(cand16, (c, sb * SB),
                                            (c + 1, sb * SB + SB)),
                                  (N_UNITS, SB))
            qb = jnp.broadcast_to(lax.slice(q16, (c, sb * SB),
                                            (c + 1, sb * SB + SB)),
                                  (N_UNITS, SB))
            wt = wt + jnp.where(units == cb, qb, jnp.bfloat16(0))
        m_acc[...] += lax.dot_general(wt, f_ref[cols, :].astype(jnp.bfloat16),
                                      (((1,), (0,)), ((), ())),
                                      preferred_element_type=jnp.float32,
                                      precision=_DEF)
    e_acc[0] += jnp.sum(jnp.log(ssum_ref[...]) + ma_ref[...])

    @pl.when(i == nsteps - 1)
    def _fin():
        m_ref[...] = m_acc[...]
        n_ref[...] = jnp.sum(np_ref[...], axis=0, keepdims=True)
        e_ref[...] = jnp.broadcast_to(e_acc[0] * (1.0 / N_SPIKES), (1, 1))


def _stage_c(qT, candT, features, ssum2, ma2, npart, chunk):
    grid = (N_CHUNK // BLK_C,)
    offc = chunk * (N_CHUNK // BLK_C)
    return pl.pallas_call(
        _stage_c_body,
        grid=grid,
        in_specs=[
            pl.BlockSpec((C_TOT, BLK_C), lambda i: (0, i)),
            pl.BlockSpec((C_TOT, BLK_C), lambda i: (0, i + offc)),
            pl.BlockSpec((BLK_C, D), lambda i: (i + offc, 0)),
            pl.BlockSpec((1, BLK_C), lambda i: (0, i)),
            pl.BlockSpec((1, BLK_C), lambda i: (0, i)),
            pl.BlockSpec((NW, N_UNITS), lambda i: (0, 0)),
        ],
        out_specs=[
            pl.BlockSpec((N_UNITS, D), lambda i: (0, 0)),
            pl.BlockSpec((1, N_UNITS), lambda i: (0, 0)),
            pl.BlockSpec((1, 1), lambda i: (0, 0)),
        ],
        out_shape=[
            jax.ShapeDtypeStruct((N_UNITS, D), jnp.float32),
            jax.ShapeDtypeStruct((1, N_UNITS), jnp.float32),
            jax.ShapeDtypeStruct((1, 1), jnp.float32),
        ],
        scratch_shapes=[
            pltpu.VMEM((N_UNITS, D), jnp.float32),
            pltpu.SMEM((1,), jnp.float32),
        ],
    )(qT, candT, features, ssum2, ma2, npart)


def kernel(features, candidates, means, log_proportions):
    candT = candidates.astype(jnp.int32).T
    lp2d = log_proportions.reshape(1, N_UNITS)
    tvs, tcs, ms, ns, es = [], [], [], [], []
    for h in range(CHUNKS):
        logits, rowsq2 = _stage_a(features, means, lp2d, h)
        tvT, tcT, qT, ssum2, ma2, npart = _stage_b(logits, candT, rowsq2, h)
        m_h, n_h, e_h = _stage_c(qT, candT, features, ssum2, ma2, npart, h)
        tvs.append(tvT)
        tcs.append(tcT)
        ms.append(m_h)
        ns.append(n_h)
        es.append(e_h)
    tvT = jnp.concatenate(tvs, axis=1)
    tcT = jnp.concatenate(tcs, axis=1)
    m_stat = sum(ms[1:], ms[0])
    n2d = sum(ns[1:], ns[0])
    e11 = sum(es[1:], es[0])
    return (tvT.T, tcT.T, n2d.reshape(N_UNITS), m_stat, e11.reshape(()))
